# baseline (speedup 1.0000x reference)
import functools

import jax
import jax.numpy as jnp
from jax import lax
from jax.experimental import pallas as pl
from jax.experimental.pallas import tpu as pltpu
from jax.experimental.pallas import tpu_sc as plsc

B, C, H, W = 8, 128, 48, 160
HW = H * W
N = 64
GH, GW = 32, 100
P = GH * GW
GWP = 128
PP = GH * GWP
NC, NS, L = 2, 16, 16
NW = NC * NS
IPW = N // NW
KCH = 4
KPAIR = KCH // 2
CPAIR = C // 2
NCHUNK = C // KCH
NGRP = P // L


def _sc_body(fm_hbm, gxy_hbm, bidx_hbm, out_hbm,
             gxy_v, idx_v, wgt_v, bidx_v,
             plane0_v, plane1_v, out0_v, out1_v,
             psem0, psem1, osem0, osem1):
    wid = lax.axis_index("s") * NC + lax.axis_index("c")
    pltpu.sync_copy(bidx_hbm, bidx_v)
    iota = lax.iota(jnp.int32, L)
    himask = jnp.full((L,), -65536, jnp.int32)

    for inst in range(IPW):
        n = wid * IPW + inst
        pltpu.sync_copy(gxy_hbm.at[pl.ds(n * (2 * P), 2 * P)], gxy_v)

        base = (n // L) * L
        bwin = bidx_v[pl.ds(base, L)]
        b = jnp.sum(jnp.where(iota == n - base, bwin, 0))

        def start_plane(c0, plane_ref, sem):
            return pltpu.async_copy(
                fm_hbm.at[pl.ds(b * CPAIR + c0 * KPAIR, KPAIR)], plane_ref, sem)

        start_plane(0, plane0_v, psem0)
        start_plane(1, plane1_v, psem1)

        @plsc.parallel_loop(0, NGRP, unroll=2)
        def ph1(g):
            s = pl.ds(g * L, L)
            gi = g * (2 * L) + iota * 2
            x = plsc.load_gather(gxy_v, [gi])
            y = plsc.load_gather(gxy_v, [gi + 1])
            ix = (x + 1.0) * (W * 0.5) - 0.5
            iy = (y + 1.0) * (H * 0.5) - 0.5
            tx = ix.astype(jnp.int32)
            ty = iy.astype(jnp.int32)
            ix0 = tx - (ix < tx.astype(jnp.float32)).astype(jnp.int32)
            iy0 = ty - (iy < ty.astype(jnp.float32)).astype(jnp.int32)
            wx1 = ix - ix0.astype(jnp.float32)
            wx0 = 1.0 - wx1
            wy1 = iy - iy0.astype(jnp.float32)
            wy0 = 1.0 - wy1
            ix1 = ix0 + 1
            iy1 = iy0 + 1
            vx0 = (ix0 >= 0) & (ix0 <= W - 1)
            vx1 = (ix1 >= 0) & (ix1 <= W - 1)
            vy0 = (iy0 >= 0) & (iy0 <= H - 1)
            vy1 = (iy1 >= 0) & (iy1 <= H - 1)
            cx0 = jnp.clip(ix0, 0, W - 1)
            cx1 = jnp.clip(ix1, 0, W - 1)
            cy0 = jnp.clip(iy0, 0, H - 1)
            cy1 = jnp.clip(iy1, 0, H - 1)
            zero = jnp.zeros((L,), jnp.float32)
            idx_v[0, s] = cy0 * W + cx0
            idx_v[1, s] = cy0 * W + cx1
            idx_v[2, s] = cy1 * W + cx0
            idx_v[3, s] = cy1 * W + cx1
            wgt_v[0, s] = jnp.where(vx0 & vy0, wx0 * wy0, zero)
            wgt_v[1, s] = jnp.where(vx1 & vy0, wx1 * wy0, zero)
            wgt_v[2, s] = jnp.where(vx0 & vy1, wx0 * wy1, zero)
            wgt_v[3, s] = jnp.where(vx1 & vy1, wx1 * wy1, zero)

        def wait_plane(plane_ref, sem):
            pltpu.make_async_copy(
                fm_hbm.at[pl.ds(0, KPAIR)], plane_ref, sem).wait()

        def process(plane_ref, out_ref):
            @plsc.parallel_loop(0, NGRP, unroll=2)
            def grp(g):
                s = pl.ds(g * L, L)
                i0 = idx_v[0, s]
                i1 = idx_v[1, s]
                i2 = idx_v[2, s]
                i3 = idx_v[3, s]
                w0 = wgt_v[0, s]
                w1 = wgt_v[1, s]
                w2 = wgt_v[2, s]
                w3 = wgt_v[3, s]
                p = g * L + iota
                oh = (p * 20972) >> 21
                ow = p - oh * GW
                for k in range(KPAIR):
                    kk = jnp.full((L,), k, jnp.int32)
                    q0 = plsc.load_gather(plane_ref, [kk, i0])
                    q1 = plsc.load_gather(plane_ref, [kk, i1])
                    q2 = plsc.load_gather(plane_ref, [kk, i2])
                    q3 = plsc.load_gather(plane_ref, [kk, i3])
                    lo0 = plsc.bitcast(q0 << 16, jnp.float32)
                    lo1 = plsc.bitcast(q1 << 16, jnp.float32)
                    lo2 = plsc.bitcast(q2 << 16, jnp.float32)
                    lo3 = plsc.bitcast(q3 << 16, jnp.float32)
                    hi0 = plsc.bitcast(q0 & himask, jnp.float32)
                    hi1 = plsc.bitcast(q1 & himask, jnp.float32)
                    hi2 = plsc.bitcast(q2 & himask, jnp.float32)
                    hi3 = plsc.bitcast(q3 & himask, jnp.float32)
                    acc_lo = lo0 * w0 + lo1 * w1 + lo2 * w2 + lo3 * w3
                    acc_hi = hi0 * w0 + hi1 * w1 + hi2 * w2 + hi3 * w3
                    plsc.store_scatter(
                        out_ref, [jnp.full((L,), k, jnp.int32), oh, ow],
                        acc_lo)
                    plsc.store_scatter(
                        out_ref, [jnp.full((L,), KPAIR + k, jnp.int32), oh, ow],
                        acc_hi)

        def start_out(c0, out_ref, sem):
            pltpu.async_copy(
                out_ref.at[pl.ds(0, KPAIR)],
                out_hbm.at[n].at[pl.ds(c0 * KPAIR, KPAIR)], sem)
            pltpu.async_copy(
                out_ref.at[pl.ds(KPAIR, KPAIR)],
                out_hbm.at[n].at[pl.ds(CPAIR + c0 * KPAIR, KPAIR)], sem)

        def wait_out(out_ref, sem):
            pltpu.make_async_copy(
                out_ref.at[pl.ds(0, KPAIR)],
                out_hbm.at[0].at[pl.ds(0, KPAIR)], sem).wait()
            pltpu.make_async_copy(
                out_ref.at[pl.ds(0, KPAIR)],
                out_hbm.at[0].at[pl.ds(0, KPAIR)], sem).wait()

        def chunk_pair(cc2, _):
            c0 = cc2 * 2
            c1 = c0 + 1

            @pl.when(cc2 > 0)
            def _():
                start_plane(c1, plane1_v, psem1)

            wait_plane(plane0_v, psem0)

            @pl.when(cc2 > 0)
            def _():
                wait_out(out0_v, osem0)

            process(plane0_v, out0_v)
            start_out(c0, out0_v, osem0)

            @pl.when(cc2 < NCHUNK // 2 - 1)
            def _():
                start_plane(c0 + 2, plane0_v, psem0)

            wait_plane(plane1_v, psem1)

            @pl.when(cc2 > 0)
            def _():
                wait_out(out1_v, osem1)

            process(plane1_v, out1_v)
            start_out(c1, out1_v, osem1)
            return 0

        lax.fori_loop(0, NCHUNK // 2, chunk_pair, 0)
        wait_out(out0_v, osem0)
        wait_out(out1_v, osem1)


@jax.jit
def _tps_align_sc(fmp, gxy, batch_idx):
    mesh = plsc.VectorSubcoreMesh(core_axis_name="c", subcore_axis_name="s")
    f = functools.partial(
        pl.kernel,
        out_type=jax.ShapeDtypeStruct((N, C, GH, GWP), jnp.float32),
        mesh=mesh,
        compiler_params=pltpu.CompilerParams(needs_layout_passes=False),
        scratch_types=[
            pltpu.VMEM((2 * P,), jnp.float32),
            pltpu.VMEM((4, P), jnp.int32),
            pltpu.VMEM((4, P), jnp.float32),
            pltpu.VMEM((N,), jnp.int32),
            pltpu.VMEM((KPAIR, HW), jnp.int32),
            pltpu.VMEM((KPAIR, HW), jnp.int32),
            pltpu.VMEM((KCH, GH, GWP), jnp.float32),
            pltpu.VMEM((KCH, GH, GWP), jnp.float32),
            pltpu.SemaphoreType.DMA,
            pltpu.SemaphoreType.DMA,
            pltpu.SemaphoreType.DMA,
            pltpu.SemaphoreType.DMA,
        ],
    )(_sc_body)
    return f(fmp, gxy, batch_idx)


def kernel(feature_map, grids, batch_idx, texts):
    grids = jax.lax.stop_gradient(grids)
    fmi = lax.bitcast_convert_type(feature_map, jnp.uint32)
    ev = fmi[:, :CPAIR]
    od = fmi[:, CPAIR:]

    def rne(u):
        return (u + jnp.uint32(0x7FFF) + ((u >> 16) & jnp.uint32(1))) >> 16

    packed = (rne(od) << 16) | rne(ev)
    fmp = lax.bitcast_convert_type(packed, jnp.int32).reshape(B * CPAIR, HW)
    gxy = grids.reshape(N * P * 2)
    outp = _tps_align_sc(fmp, gxy, batch_idx)
    feats = outp[:, :, :, :GW]
    return (feats, texts)


# speedup vs baseline: 1.0647x; 1.0647x over previous
import functools

import jax
import jax.numpy as jnp
from jax import lax
from jax.experimental import pallas as pl
from jax.experimental.pallas import tpu as pltpu
from jax.experimental.pallas import tpu_sc as plsc

B, C, H, W = 8, 128, 48, 160
HW = H * W
N = 64
GH, GW = 32, 100
P = GH * GW
GWP = 128
PP = GH * GWP
NC, NS, L = 2, 16, 16
NW = NC * NS
IPW = N // NW
KCH = 4
KPAIR = KCH // 2
CPAIR = C // 2
NCHUNK = C // KCH
NGRP = P // L


def _sc_body(fm_hbm, gxy_hbm, bidx_hbm, out_hbm,
             gxy_v, idx_v, wgt_v, addr_v, bidx_v,
             plane0_v, plane1_v, out0_v, out1_v,
             psem0, psem1, osem0, osem1):
    wid = lax.axis_index("s") * NC + lax.axis_index("c")
    pltpu.sync_copy(bidx_hbm, bidx_v)
    iota = lax.iota(jnp.int32, L)
    himask = jnp.full((L,), -65536, jnp.int32)

    for inst in range(IPW):
        n = wid * IPW + inst
        pltpu.sync_copy(gxy_hbm.at[pl.ds(n * (2 * P), 2 * P)], gxy_v)

        base = (n // L) * L
        bwin = bidx_v[pl.ds(base, L)]
        b = jnp.sum(jnp.where(iota == n - base, bwin, 0))

        def start_plane(c0, plane_ref, sem):
            return pltpu.async_copy(
                fm_hbm.at[pl.ds(b * CPAIR + c0 * KPAIR, KPAIR)], plane_ref, sem)

        start_plane(0, plane0_v, psem0)
        start_plane(1, plane1_v, psem1)

        @plsc.parallel_loop(0, NGRP, unroll=2)
        def ph1(g):
            s = pl.ds(g * L, L)
            gi = g * (2 * L) + iota * 2
            x = plsc.load_gather(gxy_v, [gi])
            y = plsc.load_gather(gxy_v, [gi + 1])
            ix = (x + 1.0) * (W * 0.5) - 0.5
            iy = (y + 1.0) * (H * 0.5) - 0.5
            tx = ix.astype(jnp.int32)
            ty = iy.astype(jnp.int32)
            ix0 = tx - (ix < tx.astype(jnp.float32)).astype(jnp.int32)
            iy0 = ty - (iy < ty.astype(jnp.float32)).astype(jnp.int32)
            wx1 = ix - ix0.astype(jnp.float32)
            wx0 = 1.0 - wx1
            wy1 = iy - iy0.astype(jnp.float32)
            wy0 = 1.0 - wy1
            ix1 = ix0 + 1
            iy1 = iy0 + 1
            vx0 = (ix0 >= 0) & (ix0 <= W - 1)
            vx1 = (ix1 >= 0) & (ix1 <= W - 1)
            vy0 = (iy0 >= 0) & (iy0 <= H - 1)
            vy1 = (iy1 >= 0) & (iy1 <= H - 1)
            cx0 = jnp.clip(ix0, 0, W - 1)
            cx1 = jnp.clip(ix1, 0, W - 1)
            cy0 = jnp.clip(iy0, 0, H - 1)
            cy1 = jnp.clip(iy1, 0, H - 1)
            zero = jnp.zeros((L,), jnp.float32)
            idx_v[0, s] = cy0 * W + cx0
            idx_v[1, s] = cy0 * W + cx1
            idx_v[2, s] = cy1 * W + cx0
            idx_v[3, s] = cy1 * W + cx1
            wgt_v[0, s] = jnp.where(vx0 & vy0, wx0 * wy0, zero)
            wgt_v[1, s] = jnp.where(vx1 & vy0, wx1 * wy0, zero)
            wgt_v[2, s] = jnp.where(vx0 & vy1, wx0 * wy1, zero)
            wgt_v[3, s] = jnp.where(vx1 & vy1, wx1 * wy1, zero)
            p = g * L + iota
            hh = (p * 20972) >> 21
            addr_v[0, s] = hh
            addr_v[1, s] = p - hh * GW

        def wait_plane(plane_ref, sem):
            pltpu.make_async_copy(
                fm_hbm.at[pl.ds(0, KPAIR)], plane_ref, sem).wait()

        def process(plane_ref, out_ref):
            @plsc.parallel_loop(0, NGRP, unroll=2)
            def grp(g):
                s = pl.ds(g * L, L)
                i0 = idx_v[0, s]
                i1 = idx_v[1, s]
                i2 = idx_v[2, s]
                i3 = idx_v[3, s]
                w0 = wgt_v[0, s]
                w1 = wgt_v[1, s]
                w2 = wgt_v[2, s]
                w3 = wgt_v[3, s]
                oh = addr_v[0, s]
                ow = addr_v[1, s]
                for k in range(KPAIR):
                    kk = jnp.full((L,), k, jnp.int32)
                    q0 = plsc.load_gather(plane_ref, [kk, i0])
                    q1 = plsc.load_gather(plane_ref, [kk, i1])
                    q2 = plsc.load_gather(plane_ref, [kk, i2])
                    q3 = plsc.load_gather(plane_ref, [kk, i3])
                    lo0 = plsc.bitcast(q0 << 16, jnp.float32)
                    lo1 = plsc.bitcast(q1 << 16, jnp.float32)
                    lo2 = plsc.bitcast(q2 << 16, jnp.float32)
                    lo3 = plsc.bitcast(q3 << 16, jnp.float32)
                    hi0 = plsc.bitcast(q0 & himask, jnp.float32)
                    hi1 = plsc.bitcast(q1 & himask, jnp.float32)
                    hi2 = plsc.bitcast(q2 & himask, jnp.float32)
                    hi3 = plsc.bitcast(q3 & himask, jnp.float32)
                    acc_lo = lo0 * w0 + lo1 * w1 + lo2 * w2 + lo3 * w3
                    acc_hi = hi0 * w0 + hi1 * w1 + hi2 * w2 + hi3 * w3
                    plsc.store_scatter(
                        out_ref, [jnp.full((L,), k, jnp.int32), oh, ow],
                        acc_lo)
                    plsc.store_scatter(
                        out_ref, [jnp.full((L,), KPAIR + k, jnp.int32), oh, ow],
                        acc_hi)

        def start_out(c0, out_ref, sem):
            pltpu.async_copy(
                out_ref.at[pl.ds(0, KPAIR)],
                out_hbm.at[n].at[pl.ds(c0 * KPAIR, KPAIR)], sem)
            pltpu.async_copy(
                out_ref.at[pl.ds(KPAIR, KPAIR)],
                out_hbm.at[n].at[pl.ds(CPAIR + c0 * KPAIR, KPAIR)], sem)

        def wait_out(out_ref, sem):
            pltpu.make_async_copy(
                out_ref.at[pl.ds(0, KPAIR)],
                out_hbm.at[0].at[pl.ds(0, KPAIR)], sem).wait()
            pltpu.make_async_copy(
                out_ref.at[pl.ds(0, KPAIR)],
                out_hbm.at[0].at[pl.ds(0, KPAIR)], sem).wait()

        def chunk_pair(cc2, _):
            c0 = cc2 * 2
            c1 = c0 + 1

            @pl.when(cc2 > 0)
            def _():
                start_plane(c1, plane1_v, psem1)

            wait_plane(plane0_v, psem0)

            @pl.when(cc2 > 0)
            def _():
                wait_out(out0_v, osem0)

            process(plane0_v, out0_v)
            start_out(c0, out0_v, osem0)

            @pl.when(cc2 < NCHUNK // 2 - 1)
            def _():
                start_plane(c0 + 2, plane0_v, psem0)

            wait_plane(plane1_v, psem1)

            @pl.when(cc2 > 0)
            def _():
                wait_out(out1_v, osem1)

            process(plane1_v, out1_v)
            start_out(c1, out1_v, osem1)
            return 0

        lax.fori_loop(0, NCHUNK // 2, chunk_pair, 0)
        wait_out(out0_v, osem0)
        wait_out(out1_v, osem1)


@jax.jit
def _tps_align_sc(fmp, gxy, batch_idx):
    mesh = plsc.VectorSubcoreMesh(core_axis_name="c", subcore_axis_name="s")
    f = functools.partial(
        pl.kernel,
        out_type=jax.ShapeDtypeStruct((N, C, GH, GWP), jnp.float32),
        mesh=mesh,
        compiler_params=pltpu.CompilerParams(needs_layout_passes=False),
        scratch_types=[
            pltpu.VMEM((2 * P,), jnp.float32),
            pltpu.VMEM((4, P), jnp.int32),
            pltpu.VMEM((4, P), jnp.float32),
            pltpu.VMEM((2, P), jnp.int32),
            pltpu.VMEM((N,), jnp.int32),
            pltpu.VMEM((KPAIR, HW), jnp.int32),
            pltpu.VMEM((KPAIR, HW), jnp.int32),
            pltpu.VMEM((KCH, GH, GWP), jnp.float32),
            pltpu.VMEM((KCH, GH, GWP), jnp.float32),
            pltpu.SemaphoreType.DMA,
            pltpu.SemaphoreType.DMA,
            pltpu.SemaphoreType.DMA,
            pltpu.SemaphoreType.DMA,
        ],
    )(_sc_body)
    return f(fmp, gxy, batch_idx)


def kernel(feature_map, grids, batch_idx, texts):
    grids = jax.lax.stop_gradient(grids)
    fmi = lax.bitcast_convert_type(feature_map, jnp.uint32)
    ev = fmi[:, :CPAIR]
    od = fmi[:, CPAIR:]

    def rne(u):
        return (u + jnp.uint32(0x7FFF) + ((u >> 16) & jnp.uint32(1))) >> 16

    packed = (rne(od) << 16) | rne(ev)
    fmp = lax.bitcast_convert_type(packed, jnp.int32).reshape(B * CPAIR, HW)
    gxy = grids.reshape(N * P * 2)
    outp = _tps_align_sc(fmp, gxy, batch_idx)
    feats = outp[:, :, :, :GW]
    return (feats, texts)
